# baseline (speedup 1.0000x reference)
import functools

import jax
import jax.numpy as jnp
from jax import lax
from jax.experimental import pallas as pl
from jax.experimental.pallas import tpu as pltpu
from jax.experimental.pallas import tpu_sc as plsc

BATCH = 16384
SEQ_LEN = 200
EMBED_DIM = 64

_info = plsc.get_sparse_core_info()
_NC = _info.num_cores
_NS = _info.num_subcores
_NW = _NC * _NS
_NBLK = 128
_BLOCKS_PW = BATCH // (_NW * _NBLK)
_L = 16
_SG = 2
_NU = SEQ_LEN // _SG

_mesh = plsc.VectorSubcoreMesh(core_axis_name="c", subcore_axis_name="s")


@functools.partial(
    pl.kernel,
    mesh=_mesh,
    out_type=jax.ShapeDtypeStruct((SEQ_LEN, EMBED_DIM, BATCH), jnp.float32),
    scratch_types=[
        pltpu.VMEM((_NBLK, SEQ_LEN), jnp.int32),
        pltpu.VMEM((SEQ_LEN * _NBLK,), jnp.int32),
        pltpu.VMEM((_SG * _NBLK, EMBED_DIM), jnp.float32),
        pltpu.VMEM((_SG * _NBLK, EMBED_DIM), jnp.float32),
        pltpu.VMEM((_SG, EMBED_DIM, _NBLK), jnp.float32),
        pltpu.VMEM((_SG, EMBED_DIM, _NBLK), jnp.float32),
        pltpu.SemaphoreType.DMA,
        pltpu.SemaphoreType.DMA,
        pltpu.SemaphoreType.DMA,
        pltpu.SemaphoreType.DMA,
        pltpu.SemaphoreType.DMA,
    ],
    compiler_params=pltpu.CompilerParams(
        use_tc_tiling_on_sc=False, needs_layout_passes=False),
)
def _gather_kernel(ids_hbm, table_hbm, y_hbm, idsblk, sidx, rows0, rows1,
                   slab0, slab1, s_ids, s_gat0, s_gat1, s_out0, s_out1):
    wid = lax.axis_index("s") * _NC + lax.axis_index("c")

    rows = (rows0, rows1)
    slab = (slab0, slab1)
    s_gat = (s_gat0, s_gat1)
    s_out = (s_out0, s_out1)

    lane = jax.lax.iota(jnp.int32, _L)
    row_idx = [lane + (_L * j) for j in range(_NBLK // _L)]

    def block(k, carry):
        b0 = (wid * _BLOCKS_PW + k) * _NBLK

        pltpu.async_copy(ids_hbm.at[pl.ds(b0, _NBLK), :], idsblk, s_ids)
        pltpu.make_async_copy(
            ids_hbm.at[pl.ds(b0, _NBLK), :], idsblk, s_ids).wait()

        @plsc.parallel_loop(0, SEQ_LEN, unroll=4)
        def tr_ids(s):
            col = jnp.full((_L,), 0, jnp.int32) + s
            for j in range(_NBLK // _L):
                sidx[pl.ds(s * _NBLK + _L * j, _L)] = plsc.load_gather(
                    idsblk, [row_idx[j], col])

        def gat_start(u, p):
            pltpu.async_copy(
                table_hbm.at[sidx.at[pl.ds(_SG * _NBLK * u, _SG * _NBLK)]],
                rows[p], s_gat[p])

        def gat_wait(u, p):
            pltpu.make_async_copy(
                table_hbm.at[sidx.at[pl.ds(_SG * _NBLK * u, _SG * _NBLK)]],
                rows[p], s_gat[p]).wait()

        def y_at(u):
            return y_hbm.at[pl.ds(_SG * u, _SG), :, pl.ds(b0, _NBLK)]

        def transpose(p):
            @plsc.parallel_loop(0, EMBED_DIM, unroll=8)
            def tr_e(e):
                col = jnp.full((_L,), 0, jnp.int32) + e
                for si in range(_SG):
                    for j in range(_NBLK // _L):
                        slab[p][si, e, pl.ds(_L * j, _L)] = plsc.load_gather(
                            rows[p], [row_idx[j] + si * _NBLK, col])

        def half(u, p, *, first=False, last=False):
            gat_wait(u, p)
            if not last:
                gat_start(u + 1, 1 - p)
            if not first:
                pltpu.make_async_copy(slab[p], y_at(u - 2), s_out[p]).wait()
            transpose(p)
            pltpu.async_copy(slab[p], y_at(u), s_out[p])

        def upair(g, c):
            half(2 * g, 0)
            half(2 * g + 1, 1)
            return c

        gat_start(0, 0)
        half(0, 0, first=True)
        half(1, 1, first=True)
        lax.fori_loop(1, _NU // 2 - 1, upair, 0)
        half(_NU - 2, 0)
        half(_NU - 1, 1, last=True)
        pltpu.make_async_copy(slab[0], y_at(_NU - 2), s_out[0]).wait()
        pltpu.make_async_copy(slab[1], y_at(_NU - 1), s_out[1]).wait()
        return carry

    lax.fori_loop(0, _BLOCKS_PW, block, 0)


def kernel(intent_ids, table):
    y = _gather_kernel(intent_ids.astype(jnp.int32), table)
    return jnp.transpose(y, (2, 0, 1))


# speedup vs baseline: 2.2953x; 2.2953x over previous
import functools

import jax
import jax.numpy as jnp
from jax import lax
from jax.experimental import pallas as pl
from jax.experimental.pallas import tpu as pltpu
from jax.experimental.pallas import tpu_sc as plsc

BATCH = 16384
SEQ_LEN = 200
EMBED_DIM = 64

_info = plsc.get_sparse_core_info()
_NC = _info.num_cores
_NS = _info.num_subcores
_NW = _NC * _NS
_NBLK = 128
_BLOCKS_PW = BATCH // (_NW * _NBLK)
_L = 16
_SG = 2
_NU = SEQ_LEN // _SG

_mesh = plsc.VectorSubcoreMesh(core_axis_name="c", subcore_axis_name="s")


@functools.partial(
    pl.kernel,
    mesh=_mesh,
    out_type=jax.ShapeDtypeStruct((SEQ_LEN, EMBED_DIM, BATCH), jnp.float32),
    scratch_types=[
        pltpu.VMEM((_NBLK, SEQ_LEN), jnp.int32),
        pltpu.VMEM((SEQ_LEN * _NBLK,), jnp.int32),
        pltpu.VMEM((_SG * _NBLK, EMBED_DIM), jnp.float32),
        pltpu.VMEM((_SG * _NBLK, EMBED_DIM), jnp.float32),
        pltpu.VMEM((_SG, EMBED_DIM, _NBLK + 1), jnp.float32),
        pltpu.VMEM((_SG, EMBED_DIM, _NBLK + 1), jnp.float32),
        pltpu.SemaphoreType.DMA,
        pltpu.SemaphoreType.DMA,
        pltpu.SemaphoreType.DMA,
        pltpu.SemaphoreType.DMA,
        pltpu.SemaphoreType.DMA,
    ],
    compiler_params=pltpu.CompilerParams(
        use_tc_tiling_on_sc=False, needs_layout_passes=False),
)
def _gather_kernel(ids_hbm, table_hbm, y_hbm, idsblk, sidx, rows0, rows1,
                   slab0, slab1, s_ids, s_gat0, s_gat1, s_out0, s_out1):
    wid = lax.axis_index("s") * _NC + lax.axis_index("c")

    rows = (rows0, rows1)
    slab = (slab0, slab1)
    s_gat = (s_gat0, s_gat1)
    s_out = (s_out0, s_out1)

    lane = jax.lax.iota(jnp.int32, _L)
    row_idx = [lane + (_L * j) for j in range(_NBLK // _L)]
    e_idx = [lane + (_L * g) for g in range(EMBED_DIM // _L)]
    si_vec = [jnp.full((_L,), si, jnp.int32) for si in range(_SG)]

    def block(k, carry):
        b0 = (wid * _BLOCKS_PW + k) * _NBLK

        pltpu.async_copy(ids_hbm.at[pl.ds(b0, _NBLK), :], idsblk, s_ids)
        pltpu.make_async_copy(
            ids_hbm.at[pl.ds(b0, _NBLK), :], idsblk, s_ids).wait()

        @plsc.parallel_loop(0, SEQ_LEN, unroll=4)
        def tr_ids(s):
            col = jnp.full((_L,), 0, jnp.int32) + s
            for j in range(_NBLK // _L):
                sidx[pl.ds(s * _NBLK + _L * j, _L)] = plsc.load_gather(
                    idsblk, [row_idx[j], col])

        def gat_start(u, p):
            pltpu.async_copy(
                table_hbm.at[sidx.at[pl.ds(_SG * _NBLK * u, _SG * _NBLK)]],
                rows[p], s_gat[p])

        def gat_wait(u, p):
            pltpu.make_async_copy(
                table_hbm.at[sidx.at[pl.ds(_SG * _NBLK * u, _SG * _NBLK)]],
                rows[p], s_gat[p]).wait()

        def y_at(u):
            return y_hbm.at[pl.ds(_SG * u, _SG), :, pl.ds(b0, _NBLK)]

        def transpose(p):
            @plsc.parallel_loop(0, _NBLK, unroll=8)
            def tr_b(b):
                for si in range(_SG):
                    sib = jnp.full((_L,), 0, jnp.int32) + b
                    for g in range(EMBED_DIM // _L):
                        v = rows[p][si * _NBLK + b, pl.ds(_L * g, _L)]
                        plsc.store_scatter(
                            slab[p], [si_vec[si], e_idx[g], sib], v)

        def half(u, p, *, first=False, last=False):
            gat_wait(u, p)
            if not last:
                gat_start(u + 1, 1 - p)
            if not first:
                pltpu.make_async_copy(
                    slab[p].at[:, :, pl.ds(0, _NBLK)], y_at(u - 2),
                    s_out[p]).wait()
            transpose(p)
            pltpu.async_copy(
                slab[p].at[:, :, pl.ds(0, _NBLK)], y_at(u), s_out[p])

        def upair(g, c):
            half(2 * g, 0)
            half(2 * g + 1, 1)
            return c

        gat_start(0, 0)
        half(0, 0, first=True)
        half(1, 1, first=True)
        lax.fori_loop(1, _NU // 2 - 1, upair, 0)
        half(_NU - 2, 0)
        half(_NU - 1, 1, last=True)
        pltpu.make_async_copy(
            slab[0].at[:, :, pl.ds(0, _NBLK)], y_at(_NU - 2), s_out[0]).wait()
        pltpu.make_async_copy(
            slab[1].at[:, :, pl.ds(0, _NBLK)], y_at(_NU - 1), s_out[1]).wait()
        return carry

    lax.fori_loop(0, _BLOCKS_PW, block, 0)


def kernel(intent_ids, table):
    y = _gather_kernel(intent_ids.astype(jnp.int32), table)
    return jnp.transpose(y, (2, 0, 1))


# speedup vs baseline: 4.8911x; 2.1309x over previous
import functools

import jax
import jax.numpy as jnp
from jax import lax
from jax.experimental import pallas as pl
from jax.experimental.pallas import tpu as pltpu
from jax.experimental.pallas import tpu_sc as plsc

BATCH = 16384
SEQ_LEN = 200
EMBED_DIM = 64

_info = plsc.get_sparse_core_info()
_NC = _info.num_cores
_NS = _info.num_subcores
_NW = _NC * _NS
_NBLK = 128
_BLOCKS_PW = BATCH // (_NW * _NBLK)
_L = 16
_SG = 2
_NU = SEQ_LEN // _SG

_mesh = plsc.VectorSubcoreMesh(core_axis_name="c", subcore_axis_name="s")


@functools.partial(
    pl.kernel,
    mesh=_mesh,
    out_type=jax.ShapeDtypeStruct(
        (SEQ_LEN, EMBED_DIM // 8, BATCH // _NBLK, 8, _NBLK), jnp.float32),
    scratch_types=[
        pltpu.VMEM((_NBLK, SEQ_LEN), jnp.int32),
        pltpu.VMEM((SEQ_LEN * _NBLK,), jnp.int32),
        pltpu.VMEM((_SG * _NBLK, EMBED_DIM), jnp.float32),
        pltpu.VMEM((_SG * _NBLK, EMBED_DIM), jnp.float32),
        pltpu.VMEM((_SG, EMBED_DIM // 8, 1, 8, _NBLK + 1), jnp.float32),
        pltpu.VMEM((_SG, EMBED_DIM // 8, 1, 8, _NBLK + 1), jnp.float32),
        pltpu.SemaphoreType.DMA,
        pltpu.SemaphoreType.DMA,
        pltpu.SemaphoreType.DMA,
        pltpu.SemaphoreType.DMA,
        pltpu.SemaphoreType.DMA,
    ],
    compiler_params=pltpu.CompilerParams(
        use_tc_tiling_on_sc=False, needs_layout_passes=False),
)
def _gather_kernel(ids_hbm, table_hbm, y_hbm, idsblk, sidx, rows0, rows1,
                   slab0, slab1, s_ids, s_gat0, s_gat1, s_out0, s_out1):
    wid = lax.axis_index("s") * _NC + lax.axis_index("c")

    rows = (rows0, rows1)
    slab = (slab0, slab1)
    s_gat = (s_gat0, s_gat1)
    s_out = (s_out0, s_out1)

    lane = jax.lax.iota(jnp.int32, _L)
    row_idx = [lane + (_L * j) for j in range(_NBLK // _L)]
    si_vec = [jnp.full((_L,), si, jnp.int32) for si in range(_SG)]
    zero16 = jnp.full((_L,), 0, jnp.int32)
    et_idx = [(lane // 8) + 2 * g for g in range(EMBED_DIM // _L)]
    ei_idx = lane % 8

    def block(k, carry):
        b0 = (wid * _BLOCKS_PW + k) * _NBLK

        pltpu.async_copy(ids_hbm.at[pl.ds(b0, _NBLK), :], idsblk, s_ids)
        pltpu.make_async_copy(
            ids_hbm.at[pl.ds(b0, _NBLK), :], idsblk, s_ids).wait()

        @plsc.parallel_loop(0, SEQ_LEN, unroll=4)
        def tr_ids(s):
            col = jnp.full((_L,), 0, jnp.int32) + s
            for j in range(_NBLK // _L):
                sidx[pl.ds(s * _NBLK + _L * j, _L)] = plsc.load_gather(
                    idsblk, [row_idx[j], col])

        def gat_start(u, p):
            pltpu.async_copy(
                table_hbm.at[sidx.at[pl.ds(_SG * _NBLK * u, _SG * _NBLK)]],
                rows[p], s_gat[p])

        def gat_wait(u, p):
            pltpu.make_async_copy(
                table_hbm.at[sidx.at[pl.ds(_SG * _NBLK * u, _SG * _NBLK)]],
                rows[p], s_gat[p]).wait()

        bt = wid * _BLOCKS_PW + k
        def y_at(u):
            return y_hbm.at[pl.ds(_SG * u, _SG), :, pl.ds(bt, 1), :, :]

        def transpose(p):
            @plsc.parallel_loop(0, _NBLK, unroll=8)
            def tr_b(b):
                for si in range(_SG):
                    sib = jnp.full((_L,), 0, jnp.int32) + b
                    for g in range(EMBED_DIM // _L):
                        v = rows[p][si * _NBLK + b, pl.ds(_L * g, _L)]
                        plsc.store_scatter(
                            slab[p],
                            [si_vec[si], et_idx[g], zero16, ei_idx, sib], v)

        def half(u, p, *, first=False, last=False):
            gat_wait(u, p)
            if not last:
                gat_start(u + 1, 1 - p)
            if not first:
                pltpu.make_async_copy(
                    slab[p].at[:, :, :, :, pl.ds(0, _NBLK)], y_at(u - 2),
                    s_out[p]).wait()
            transpose(p)
            pltpu.async_copy(
                slab[p].at[:, :, :, :, pl.ds(0, _NBLK)], y_at(u), s_out[p])

        def upair(g, c):
            half(2 * g, 0)
            half(2 * g + 1, 1)
            return c

        gat_start(0, 0)
        half(0, 0, first=True)
        half(1, 1, first=True)
        lax.fori_loop(1, _NU // 2 - 1, upair, 0)
        half(_NU - 2, 0)
        half(_NU - 1, 1, last=True)
        pltpu.make_async_copy(
            slab[0].at[:, :, :, :, pl.ds(0, _NBLK)], y_at(_NU - 2), s_out[0]).wait()
        pltpu.make_async_copy(
            slab[1].at[:, :, :, :, pl.ds(0, _NBLK)], y_at(_NU - 1), s_out[1]).wait()
        return carry

    lax.fori_loop(0, _BLOCKS_PW, block, 0)


def kernel(intent_ids, table):
    y = _gather_kernel(intent_ids.astype(jnp.int32), table)
    return jnp.transpose(y, (2, 4, 0, 1, 3)).reshape(BATCH, SEQ_LEN, EMBED_DIM)
